# baseline (speedup 1.0000x reference)
import functools
import math

import jax
import jax.numpy as jnp
from jax.experimental import pallas as pl
from jax.experimental.pallas import tpu as pltpu

_DEF = jax.lax.Precision.DEFAULT


def _body(H, kk, scale, x_ref, wr_ref, br_ref, wo_ref, bout_ref, s_ref,
          o_ref, bias_scr, sc_buf, v_buf):
    h = pl.program_id(1)

    @pl.when(h == 0)
    def _():
        bias_scr[...] = jnp.log(s_ref[0] + 1e-8)

    hc = jnp.minimum(h, H - 1)
    xb = x_ref[0]
    wq = wr_ref[pl.ds(hc, 1)][0]
    wk = wr_ref[pl.ds(H + hc, 1)][0]
    wv = wr_ref[pl.ds(2 * H + hc, 1)][0]
    bq = br_ref[pl.ds(hc, 1)][0]
    bk = br_ref[pl.ds(H + hc, 1)][0]
    bv = br_ref[pl.ds(2 * H + hc, 1)][0]

    dn_t = (((1,), (1,)), ((), ()))
    q = jax.lax.dot_general(xb, wq, dn_t, precision=_DEF) + bq[None, :]
    k = jax.lax.dot_general(xb, wk, dn_t, precision=_DEF) + bk[None, :]
    v = jax.lax.dot_general(xb, wv, dn_t, precision=_DEF) + bv[None, :]

    sc = jax.lax.dot_general(k, q, dn_t, precision=_DEF) * scale
    sc_buf[h % 2] = sc + bias_scr[...]
    v_buf[h % 2] = v

    scores = sc_buf[(h + 1) % 2][...]
    vp = v_buf[(h + 1) % 2][...]

    m = jnp.max(scores, axis=0, keepdims=True)
    lo = jnp.min(scores, axis=0, keepdims=True)
    hi = m
    for _ in range(13):
        mid = 0.5 * (lo + hi)
        cnt = jnp.sum((scores >= mid).astype(jnp.float32), axis=0,
                      keepdims=True)
        ge = cnt >= kk
        lo = jnp.where(ge, mid, lo)
        hi = jnp.where(ge, hi, mid)

    c_hi = jnp.sum((scores >= hi).astype(jnp.int32), axis=0, keepdims=True)
    r = kk - c_hi
    done0 = ((r <= 0) | (h == 0)).astype(jnp.int32)
    thr0 = jnp.where(done0 == 1, hi, lo)

    def ext_cond(state):
        done, _, _, _ = state
        return jnp.min(done) == 0

    def ext_body(state):
        done, r, thr, ub = state
        cand = (scores >= lo) & (scores < ub)
        mc = jnp.max(jnp.where(cand, scores, -jnp.inf), axis=0,
                     keepdims=True)
        c_m = jnp.sum((scores == mc).astype(jnp.int32), axis=0,
                      keepdims=True)
        active = done == 0
        take = active & (r <= c_m)
        thr = jnp.where(take, mc, thr)
        done = jnp.where(take, 1, done)
        cont = active & jnp.logical_not(take)
        r = jnp.where(cont, r - c_m, r)
        ub = jnp.where(cont, mc, ub)
        return done, r, thr, ub

    _, _, thr, _ = jax.lax.while_loop(
        ext_cond, ext_body, (done0, r, thr0, hi))

    sel = scores >= thr
    p = jnp.where(sel, jnp.exp(scores - m), 0.0)
    z = jnp.sum(p, axis=0, keepdims=True)
    attn_t = p / z

    dn_n = (((1,), (0,)), ((), ()))
    dn_c0 = (((0,), (0,)), ((), ()))
    hp = jnp.maximum(h - 1, 0)
    o = jax.lax.dot_general(attn_t, vp, dn_c0, precision=_DEF)
    proj = jax.lax.dot_general(o, wo_ref[pl.ds(hp, 1)][0], dn_n,
                               precision=_DEF)

    @pl.when(h >= 1)
    def _():
        first = h <= 1
        o_ref[0] = jnp.where(first, proj + bout_ref[0][None, :],
                             o_ref[0] + proj)


def kernel(x, Wqkv, bqkv, Wout, bout, S_struc):
    L, B, D = x.shape
    H = S_struc.shape[0]
    hd = D // H
    kk = max(1, int(0.1 * L))
    scale = 1.0 / math.sqrt(hd)

    Wr = Wqkv.reshape(3 * H, hd, D)
    br = bqkv.reshape(3 * H, hd)
    Wo = jnp.transpose(Wout.reshape(D, H, hd), (1, 2, 0))
    bo = bout.reshape(1, D)

    body = functools.partial(_body, H, kk, scale)
    xt = jnp.transpose(x, (1, 0, 2))

    out = pl.pallas_call(
        body,
        grid=(B, H + 1),
        in_specs=[
            pl.BlockSpec((1, L, D), lambda b, h: (b, 0, 0)),
            pl.BlockSpec((3 * H, hd, D), lambda b, h: (0, 0, 0)),
            pl.BlockSpec((3 * H, hd), lambda b, h: (0, 0)),
            pl.BlockSpec((H, hd, D), lambda b, h: (0, 0, 0)),
            pl.BlockSpec((1, D), lambda b, h: (0, 0)),
            pl.BlockSpec((1, L, L), lambda b, h: (b, 0, 0)),
        ],
        out_specs=pl.BlockSpec((1, L, D), lambda b, h: (b, 0, 0)),
        out_shape=jax.ShapeDtypeStruct((B, L, D), jnp.float32),
        scratch_shapes=[
            pltpu.VMEM((L, L), jnp.float32),
            pltpu.VMEM((2, L, L), jnp.float32),
            pltpu.VMEM((2, L, hd), jnp.float32),
        ],
        compiler_params=pltpu.CompilerParams(
            dimension_semantics=("arbitrary", "arbitrary")),
    )(xt, Wr, br, Wo, bo, jnp.transpose(S_struc, (0, 2, 1)))
    return jnp.transpose(out, (1, 0, 2))


# speedup vs baseline: 1.2495x; 1.2495x over previous
import functools
import math

import jax
import jax.numpy as jnp
from jax.experimental import pallas as pl
from jax.experimental.pallas import tpu as pltpu

_HIGH = jax.lax.Precision.HIGHEST


def _body(H, kk, scale, x_ref, wr_ref, br_ref, wo_ref, bout_ref, s_ref,
          o_ref, bias_scr):
    h = pl.program_id(1)

    @pl.when(h == 0)
    def _():
        bias_scr[...] = jnp.log(s_ref[0] + 1e-8)

    xb = x_ref[0]
    wq = wr_ref[pl.ds(h, 1)][0]
    wk = wr_ref[pl.ds(H + h, 1)][0]
    wv = wr_ref[pl.ds(2 * H + h, 1)][0]
    bq = br_ref[pl.ds(h, 1)][0]
    bk = br_ref[pl.ds(H + h, 1)][0]
    bv = br_ref[pl.ds(2 * H + h, 1)][0]

    dn_t = (((1,), (1,)), ((), ()))
    q = jax.lax.dot_general(xb, wq, dn_t, precision=jax.lax.Precision.DEFAULT) + bq[None, :]
    k = jax.lax.dot_general(xb, wk, dn_t, precision=jax.lax.Precision.DEFAULT) + bk[None, :]
    v = jax.lax.dot_general(xb, wv, dn_t, precision=jax.lax.Precision.DEFAULT) + bv[None, :]

    scores = jax.lax.dot_general(k, q, dn_t, precision=jax.lax.Precision.DEFAULT) * scale
    scores = scores + bias_scr[...]

    m = jnp.max(scores, axis=0, keepdims=True)
    lo0 = jnp.min(scores, axis=0, keepdims=True)

    def step(_, lh):
        lo, hi = lh
        mid = 0.5 * (lo + hi)
        cnt = jnp.sum((scores >= mid).astype(jnp.float32), axis=0,
                      keepdims=True)
        ge = cnt >= kk
        return jnp.where(ge, mid, lo), jnp.where(ge, hi, mid)

    lo, hi = jax.lax.fori_loop(0, 13, step, (lo0, m))

    c_hi = jnp.sum((scores >= hi).astype(jnp.int32), axis=0, keepdims=True)
    r = kk - c_hi
    done0 = (r <= 0).astype(jnp.int32)
    thr0 = jnp.where(done0 == 1, hi, lo)

    def ext_cond(state):
        done, _, _, _ = state
        return jnp.min(done) == 0

    def ext_body(state):
        done, r, thr, ub = state
        cand = (scores >= lo) & (scores < ub)
        mc = jnp.max(jnp.where(cand, scores, -jnp.inf), axis=0,
                     keepdims=True)
        c_m = jnp.sum((scores == mc).astype(jnp.int32), axis=0,
                      keepdims=True)
        active = done == 0
        take = active & (r <= c_m)
        thr = jnp.where(take, mc, thr)
        done = jnp.where(take, 1, done)
        cont = active & jnp.logical_not(take)
        r = jnp.where(cont, r - c_m, r)
        ub = jnp.where(cont, mc, ub)
        return done, r, thr, ub

    _, _, thr, _ = jax.lax.while_loop(
        ext_cond, ext_body, (done0, r, thr0, hi))

    sel = scores >= thr
    p = jnp.where(sel, jnp.exp(scores - m), 0.0)

    vx = jnp.concatenate([v, jnp.ones((v.shape[0], 1), jnp.float32)],
                         axis=1)
    dn_n = (((1,), (0,)), ((), ()))
    dn_c0 = (((0,), (0,)), ((), ()))
    hd = v.shape[1]
    o_ext = jax.lax.dot_general(p, vx, dn_c0,
                                precision=jax.lax.Precision.DEFAULT)
    o = o_ext[:, :hd] * (1.0 / o_ext[:, hd:hd + 1])
    proj = jax.lax.dot_general(o, wo_ref[pl.ds(h, 1)][0], dn_n,
                               precision=jax.lax.Precision.DEFAULT)

    @pl.when(h == 0)
    def _():
        o_ref[0] = proj + bout_ref[0][None, :]

    @pl.when(h != 0)
    def _():
        o_ref[0] = o_ref[0] + proj


def kernel(x, Wqkv, bqkv, Wout, bout, S_struc):
    L, B, D = x.shape
    H = S_struc.shape[0]
    hd = D // H
    kk = max(1, int(0.1 * L))
    scale = 1.0 / math.sqrt(hd)

    Wr = Wqkv.reshape(3 * H, hd, D)
    br = bqkv.reshape(3 * H, hd)
    Wo = jnp.transpose(Wout.reshape(D, H, hd), (1, 2, 0))
    bo = bout.reshape(1, D)

    body = functools.partial(_body, H, kk, scale)
    xt = jnp.transpose(x, (1, 0, 2))

    out = pl.pallas_call(
        body,
        grid=(B, H),
        in_specs=[
            pl.BlockSpec((1, L, D), lambda b, h: (b, 0, 0)),
            pl.BlockSpec((3 * H, hd, D), lambda b, h: (0, 0, 0)),
            pl.BlockSpec((3 * H, hd), lambda b, h: (0, 0)),
            pl.BlockSpec((H, hd, D), lambda b, h: (0, 0, 0)),
            pl.BlockSpec((1, D), lambda b, h: (0, 0)),
            pl.BlockSpec((1, L, L), lambda b, h: (b, 0, 0)),
        ],
        out_specs=pl.BlockSpec((1, L, D), lambda b, h: (b, 0, 0)),
        out_shape=jax.ShapeDtypeStruct((B, L, D), jnp.float32),
        scratch_shapes=[pltpu.VMEM((L, L), jnp.float32)],
        compiler_params=pltpu.CompilerParams(
            dimension_semantics=("arbitrary", "arbitrary")),
    )(xt, Wr, br, Wo, bo, jnp.transpose(S_struc, (0, 2, 1)))
    return jnp.transpose(out, (1, 0, 2))
